# Initial kernel scaffold; baseline (speedup 1.0000x reference)
import jax
import jax.numpy as jnp
from jax.experimental import pallas as pl


def kernel(x, edge_index, batch, W1, b1, W2, b2, W3, b3, Wfc, bfc):
    raise NotImplementedError("write your pallas kernel here")


# speedup vs baseline: 6.9883x; 6.9883x over previous
import functools

import jax
import jax.numpy as jnp
from jax import lax
from jax.experimental import pallas as pl
from jax.experimental.pallas import tpu as pltpu
from jax.experimental.pallas import tpu_sc as plsc

N = 10000
D = 256
H = 256
C = 2
G = 64
E = 160000

NPAD = 10240
NTILE = 16
CH = 80
CK = 128
EP = NTILE * CH * CK
DUMP = NPAD - 8
RPT = NPAD // NTILE
HALF = 128
BM = 512


def _sc_degree(dst3):
    mesh = plsc.VectorSubcoreMesh(core_axis_name="c", subcore_axis_name="s")

    @functools.partial(
        pl.kernel,
        mesh=mesh,
        out_type=(
            jax.ShapeDtypeStruct((NPAD, 16), jnp.float32),
            jax.ShapeDtypeStruct((NPAD, 16), jnp.float32),
        ),
        scratch_types=[
            pltpu.VMEM((CH, CK), jnp.int32),
            pltpu.VMEM((CK, 16), jnp.float32),
            pltpu.VMEM((CK, 16), jnp.float32),
            pltpu.VMEM_SHARED((NPAD, 16), jnp.float32),
        ],
    )
    def k(dst_hbm, deg0_hbm, deg1_hbm, dst_slab, ones_v, zero_v, acc):
        c = lax.axis_index("c")
        s = lax.axis_index("s")
        pltpu.sync_copy(dst_hbm.at[s], dst_slab)

        def fill(i, _):
            ones_v[i, :] = jnp.ones((16,), jnp.float32)
            zero_v[i, :] = jnp.zeros((16,), jnp.float32)
            return 0

        lax.fori_loop(0, CK, fill, 0)
        for kk in range(RPT // CK):
            pltpu.sync_copy(zero_v, acc.at[pl.ds(s * RPT + kk * CK, CK)])
        plsc.subcore_barrier()

        base = c * (CH // 2)

        def body(j, _):
            pltpu.sync_copy(ones_v, acc.at[dst_slab.at[base + j]], add=True)
            return 0

        lax.fori_loop(0, CH // 2, body, 0)
        plsc.subcore_barrier()

        @pl.when(c == 0)
        def _():
            pltpu.sync_copy(acc.at[pl.ds(s * RPT, RPT)],
                            deg0_hbm.at[pl.ds(s * RPT, RPT)])

        @pl.when(c == 1)
        def _():
            pltpu.sync_copy(acc.at[pl.ds(s * RPT, RPT)],
                            deg1_hbm.at[pl.ds(s * RPT, RPT)])

    return k(dst3)


def _sc_aggregate(yA, yB, src3, dst3):
    mesh = plsc.VectorSubcoreMesh(core_axis_name="c", subcore_axis_name="s")

    @functools.partial(
        pl.kernel,
        mesh=mesh,
        out_type=(
            jax.ShapeDtypeStruct((NPAD, HALF), jnp.float32),
            jax.ShapeDtypeStruct((NPAD, HALF), jnp.float32),
        ),
        scratch_types=[
            pltpu.VMEM((CH // 2, CK), jnp.int32),
            pltpu.VMEM((CH // 2, CK), jnp.int32),
            pltpu.VMEM((CK, HALF), jnp.float32),
            pltpu.VMEM((CK, HALF), jnp.float32),
            pltpu.VMEM_SHARED((NPAD, HALF), jnp.float32),
            pltpu.SemaphoreType.DMA,
            pltpu.SemaphoreType.DMA,
        ],
    )
    def k(yA_hbm, yB_hbm, src_hbm, dst_hbm, aggA_hbm, aggB_hbm,
          src_slab, dst_slab, rows0, rows1, acc, sem0, sem1):
        c = lax.axis_index("c")
        s = lax.axis_index("s")

        def zfill(i, _):
            for l in range(HALF // 16):
                rows0[i, pl.ds(l * 16, 16)] = jnp.zeros((16,), jnp.float32)
            return 0

        lax.fori_loop(0, CK, zfill, 0)
        for kk in range(RPT // CK):
            pltpu.sync_copy(rows0, acc.at[pl.ds(s * RPT + kk * CK, CK)])
        plsc.subcore_barrier()

        def run(y_hbm, out_hbm):
            hc = CH // 2
            for phase in range(2):
                pltpu.sync_copy(src_hbm.at[s, pl.ds(phase * hc, hc)], src_slab)
                pltpu.sync_copy(dst_hbm.at[s, pl.ds(phase * hc, hc)], dst_slab)

                def body(j2, _):
                    j = j2 * 2
                    cp0 = pltpu.async_copy(y_hbm.at[src_slab.at[j]], rows0,
                                           sem0)
                    cp1 = pltpu.async_copy(y_hbm.at[src_slab.at[j + 1]], rows1,
                                           sem1)
                    cp0.wait()
                    pltpu.sync_copy(rows0, acc.at[dst_slab.at[j]], add=True)
                    cp1.wait()
                    pltpu.sync_copy(rows1, acc.at[dst_slab.at[j + 1]], add=True)
                    return 0

                lax.fori_loop(0, hc // 2, body, 0)
            plsc.subcore_barrier()
            pltpu.sync_copy(acc.at[pl.ds(s * RPT, RPT)],
                            out_hbm.at[pl.ds(s * RPT, RPT)])

        @pl.when(c == 0)
        def _():
            run(yA_hbm, aggA_hbm)

        @pl.when(c == 1)
        def _():
            run(yB_hbm, aggB_hbm)

    return k(yA, yB, src3, dst3)


def _dinv_of(d_ref):
    return lax.rsqrt(d_ref[:, 0:1] + 1.0)


def _mm_t(a, w):
    return lax.dot_general(a, w, (((1,), (1,)), ((), ())),
                           preferred_element_type=jnp.float32,
                           precision=lax.Precision.HIGHEST)


def _tc_layer1(xp, W1, deg):
    def body(x_ref, w_ref, d_ref, yA_ref, yB_ref):
        dinv = _dinv_of(d_ref)
        y = _mm_t(x_ref[...], w_ref[...]) * dinv
        yA_ref[...] = y[:, :HALF]
        yB_ref[...] = y[:, HALF:]

    return pl.pallas_call(
        body,
        grid=(NPAD // BM,),
        in_specs=[
            pl.BlockSpec((BM, D), lambda i: (i, 0)),
            pl.BlockSpec((H, D), lambda i: (0, 0)),
            pl.BlockSpec((BM, 16), lambda i: (i, 0)),
        ],
        out_specs=[
            pl.BlockSpec((BM, HALF), lambda i: (i, 0)),
            pl.BlockSpec((BM, HALF), lambda i: (i, 0)),
        ],
        out_shape=[jax.ShapeDtypeStruct((NPAD, HALF), jnp.float32)] * 2,
    )(xp, W1, deg)


def _tc_layer(aggA, aggB, yA, yB, W, b2d, deg):
    def body(aA_ref, aB_ref, yA_ref, yB_ref, w_ref, b_ref, d_ref,
             oA_ref, oB_ref):
        dinv = _dinv_of(d_ref)
        u = jnp.concatenate(
            [aA_ref[...] + yA_ref[...], aB_ref[...] + yB_ref[...]], axis=1)
        xn = jnp.maximum(u * dinv + b_ref[...], 0.0)
        y = _mm_t(xn, w_ref[...]) * dinv
        oA_ref[...] = y[:, :HALF]
        oB_ref[...] = y[:, HALF:]

    half_spec = pl.BlockSpec((BM, HALF), lambda i: (i, 0))
    return pl.pallas_call(
        body,
        grid=(NPAD // BM,),
        in_specs=[
            half_spec, half_spec, half_spec, half_spec,
            pl.BlockSpec((H, H), lambda i: (0, 0)),
            pl.BlockSpec((1, H), lambda i: (0, 0)),
            pl.BlockSpec((BM, 16), lambda i: (i, 0)),
        ],
        out_specs=[half_spec, half_spec],
        out_shape=[jax.ShapeDtypeStruct((NPAD, HALF), jnp.float32)] * 2,
    )(aggA, aggB, yA, yB, W, b2d, deg)


def _tc_final(aggA, aggB, yA, yB, deg, b3_2d, batch3, wfc_p, bfc_p):
    nblk = NPAD // BM

    def body(aA_ref, aB_ref, yA_ref, yB_ref, d_ref, b_ref, bt_ref,
             wfc_ref, bfc_ref, out_ref, acc, cnt):
        i = pl.program_id(0)

        @pl.when(i == 0)
        def _():
            acc[...] = jnp.zeros_like(acc)
            cnt[...] = jnp.zeros_like(cnt)

        dinv = _dinv_of(d_ref)
        u = jnp.concatenate(
            [aA_ref[...] + yA_ref[...], aB_ref[...] + yB_ref[...]], axis=1)
        h = jnp.maximum(u * dinv + b_ref[...], 0.0)
        bt = jnp.reshape(bt_ref[...], (1, BM))
        ohT = (lax.broadcasted_iota(jnp.int32, (G, 1), 0) == bt
               ).astype(jnp.float32)
        acc[...] += lax.dot_general(ohT, h, (((1,), (0,)), ((), ())),
                                    preferred_element_type=jnp.float32,
                                    precision=lax.Precision.HIGHEST)
        cnt[...] += lax.dot_general(ohT, jnp.ones((BM, 128), jnp.float32),
                                    (((1,), (0,)), ((), ())),
                                    preferred_element_type=jnp.float32,
                                    precision=lax.Precision.HIGHEST)

        @pl.when(i == nblk - 1)
        def _():
            pooled = acc[...] / jnp.maximum(cnt[:, 0:1], 1.0)
            out_ref[...] = _mm_t(pooled, wfc_ref[...]) + bfc_ref[...]

    half_spec = pl.BlockSpec((BM, HALF), lambda i: (i, 0))
    return pl.pallas_call(
        body,
        grid=(nblk,),
        in_specs=[
            half_spec, half_spec, half_spec, half_spec,
            pl.BlockSpec((BM, 16), lambda i: (i, 0)),
            pl.BlockSpec((1, H), lambda i: (0, 0)),
            pl.BlockSpec((1, 1, BM), lambda i: (i, 0, 0)),
            pl.BlockSpec((128, H), lambda i: (0, 0)),
            pl.BlockSpec((1, 128), lambda i: (0, 0)),
        ],
        out_specs=pl.BlockSpec((G, 128), lambda i: (0, 0)),
        out_shape=jax.ShapeDtypeStruct((G, 128), jnp.float32),
        scratch_shapes=[
            pltpu.VMEM((G, H), jnp.float32),
            pltpu.VMEM((G, 128), jnp.float32),
        ],
    )(aggA, aggB, yA, yB, deg, b3_2d, batch3, wfc_p, bfc_p)


def kernel(x, edge_index, batch, W1, b1, W2, b2, W3, b3, Wfc, bfc):
    src = edge_index[0]
    dst = edge_index[1]
    pad_e = EP - E
    srcp = jnp.concatenate(
        [src, jnp.zeros((pad_e,), jnp.int32)]).reshape(NTILE, CH, CK)
    dstp = jnp.concatenate(
        [dst, jnp.full((pad_e,), DUMP, jnp.int32)]).reshape(NTILE, CH, CK)
    xp = jnp.concatenate([x, jnp.zeros((NPAD - N, D), jnp.float32)], axis=0)
    batchp = jnp.concatenate(
        [batch, jnp.full((NPAD - N,), G, jnp.int32)]).reshape(NPAD // BM, 1, BM)
    b1r = b1.reshape(1, H)
    b2r = b2.reshape(1, H)
    b3r = b3.reshape(1, H)
    wfc_p = jnp.zeros((128, H), jnp.float32).at[:C].set(Wfc)
    bfc_p = jnp.zeros((1, 128), jnp.float32).at[0, :C].set(bfc)

    deg0, deg1 = _sc_degree(dstp)
    deg = deg0 + deg1
    yA, yB = _tc_layer1(xp, W1, deg)
    aggA, aggB = _sc_aggregate(yA, yB, srcp, dstp)
    yA, yB = _tc_layer(aggA, aggB, yA, yB, W2, b1r, deg)
    aggA, aggB = _sc_aggregate(yA, yB, srcp, dstp)
    yA, yB = _tc_layer(aggA, aggB, yA, yB, W3, b2r, deg)
    aggA, aggB = _sc_aggregate(yA, yB, srcp, dstp)
    out = _tc_final(aggA, aggB, yA, yB, deg, b3r, batchp, wfc_p, bfc_p)
    return out[:, :C]


# speedup vs baseline: 7.9897x; 1.1433x over previous
import functools

import jax
import jax.numpy as jnp
from jax import lax
from jax.experimental import pallas as pl
from jax.experimental.pallas import tpu as pltpu
from jax.experimental.pallas import tpu_sc as plsc

N = 10000
D = 256
H = 256
C = 2
G = 64
E = 160000

NPAD = 10240
NTILE = 16
CH = 80
CK = 128
EP = NTILE * CH * CK
DUMP = NPAD - 8
RPT = NPAD // NTILE
HALF = 128
BM = 512


def _sc_degree(dst3):
    mesh = plsc.VectorSubcoreMesh(core_axis_name="c", subcore_axis_name="s")

    @functools.partial(
        pl.kernel,
        mesh=mesh,
        out_type=(
            jax.ShapeDtypeStruct((NPAD, 16), jnp.float32),
            jax.ShapeDtypeStruct((NPAD, 16), jnp.float32),
        ),
        scratch_types=[
            pltpu.VMEM((CH, CK), jnp.int32),
            pltpu.VMEM((CK, 16), jnp.float32),
            pltpu.VMEM((CK, 16), jnp.float32),
            pltpu.VMEM_SHARED((NPAD, 16), jnp.float32),
        ],
    )
    def k(dst_hbm, deg0_hbm, deg1_hbm, dst_slab, ones_v, zero_v, acc):
        c = lax.axis_index("c")
        s = lax.axis_index("s")
        pltpu.sync_copy(dst_hbm.at[s], dst_slab)

        def fill(i, _):
            ones_v[i, :] = jnp.ones((16,), jnp.float32)
            zero_v[i, :] = jnp.zeros((16,), jnp.float32)
            return 0

        lax.fori_loop(0, CK, fill, 0)
        for kk in range(RPT // CK):
            pltpu.sync_copy(zero_v, acc.at[pl.ds(s * RPT + kk * CK, CK)])
        plsc.subcore_barrier()

        base = c * (CH // 2)

        def body(j, _):
            pltpu.sync_copy(ones_v, acc.at[dst_slab.at[base + j]], add=True)
            return 0

        lax.fori_loop(0, CH // 2, body, 0)
        plsc.subcore_barrier()

        @pl.when(c == 0)
        def _():
            pltpu.sync_copy(acc.at[pl.ds(s * RPT, RPT)],
                            deg0_hbm.at[pl.ds(s * RPT, RPT)])

        @pl.when(c == 1)
        def _():
            pltpu.sync_copy(acc.at[pl.ds(s * RPT, RPT)],
                            deg1_hbm.at[pl.ds(s * RPT, RPT)])

    return k(dst3)


def _sc_aggregate(yA, yB, src3, dst3):
    mesh = plsc.VectorSubcoreMesh(core_axis_name="c", subcore_axis_name="s")

    @functools.partial(
        pl.kernel,
        mesh=mesh,
        out_type=(
            jax.ShapeDtypeStruct((NPAD, HALF), jnp.float32),
            jax.ShapeDtypeStruct((NPAD, HALF), jnp.float32),
        ),
        scratch_types=[
            pltpu.VMEM((CH // 2, CK), jnp.int32),
            pltpu.VMEM((CH // 2, CK), jnp.int32),
            pltpu.VMEM((CK, HALF), jnp.float32),
            pltpu.VMEM((CK, HALF), jnp.float32),
            pltpu.VMEM_SHARED((NPAD, HALF), jnp.float32),
            pltpu.SemaphoreType.DMA,
            pltpu.SemaphoreType.DMA,
            pltpu.SemaphoreType.DMA,
            pltpu.SemaphoreType.DMA,
        ],
    )
    def k(yA_hbm, yB_hbm, src_hbm, dst_hbm, aggA_hbm, aggB_hbm,
          src_slab, dst_slab, rows0, rows1, acc, gs0, gs1, ss0, ss1):
        c = lax.axis_index("c")
        s = lax.axis_index("s")

        def zfill(i, _):
            for l in range(HALF // 16):
                rows0[i, pl.ds(l * 16, 16)] = jnp.zeros((16,), jnp.float32)
            return 0

        lax.fori_loop(0, CK, zfill, 0)
        for kk in range(RPT // CK):
            pltpu.sync_copy(rows0, acc.at[pl.ds(s * RPT + kk * CK, CK)])
        plsc.subcore_barrier()

        def run(y_hbm, out_hbm):
            hc = CH // 2

            def g(buf, j, sem):
                pltpu.async_copy(y_hbm.at[src_slab.at[j]], buf, sem)

            def gw(buf, j, sem):
                pltpu.make_async_copy(y_hbm.at[src_slab.at[j]], buf,
                                      sem).wait()

            def sc(buf, j, sem):
                pltpu.async_copy(buf, acc.at[dst_slab.at[j]], sem, add=True)

            def scw(buf, j, sem):
                pltpu.make_async_copy(buf, acc.at[dst_slab.at[j]], sem).wait()

            for phase in range(2):
                pltpu.sync_copy(src_hbm.at[s, pl.ds(phase * hc, hc)], src_slab)
                pltpu.sync_copy(dst_hbm.at[s, pl.ds(phase * hc, hc)], dst_slab)
                g(rows0, 0, gs0)
                g(rows1, 1, gs1)

                def body(k2, _):
                    j = k2 * 2
                    gw(rows0, j, gs0)
                    pltpu.sync_copy(rows0, acc.at[dst_slab.at[j]], add=True)
                    g(rows0, j + 2, gs0)
                    gw(rows1, j + 1, gs1)
                    pltpu.sync_copy(rows1, acc.at[dst_slab.at[j + 1]],
                                    add=True)
                    g(rows1, j + 3, gs1)
                    return 0

                lax.fori_loop(0, hc // 2 - 1, body, 0)
                jlast = hc - 2
                gw(rows0, jlast, gs0)
                pltpu.sync_copy(rows0, acc.at[dst_slab.at[jlast]], add=True)
                gw(rows1, jlast + 1, gs1)
                pltpu.sync_copy(rows1, acc.at[dst_slab.at[jlast + 1]],
                                add=True)
            plsc.subcore_barrier()
            pltpu.sync_copy(acc.at[pl.ds(s * RPT, RPT)],
                            out_hbm.at[pl.ds(s * RPT, RPT)])

        @pl.when(c == 0)
        def _():
            run(yA_hbm, aggA_hbm)

        @pl.when(c == 1)
        def _():
            run(yB_hbm, aggB_hbm)

    return k(yA, yB, src3, dst3)


def _dinv_of(d_ref):
    return lax.rsqrt(d_ref[:, 0:1] + 1.0)


def _mm_t(a, w):
    return lax.dot_general(a, w, (((1,), (1,)), ((), ())),
                           preferred_element_type=jnp.float32,
                           precision=lax.Precision.HIGHEST)


def _tc_layer1(xp, W1, deg):
    def body(x_ref, w_ref, d_ref, yA_ref, yB_ref):
        dinv = _dinv_of(d_ref)
        y = _mm_t(x_ref[...], w_ref[...]) * dinv
        yA_ref[...] = y[:, :HALF]
        yB_ref[...] = y[:, HALF:]

    return pl.pallas_call(
        body,
        grid=(NPAD // BM,),
        in_specs=[
            pl.BlockSpec((BM, D), lambda i: (i, 0)),
            pl.BlockSpec((H, D), lambda i: (0, 0)),
            pl.BlockSpec((BM, 16), lambda i: (i, 0)),
        ],
        out_specs=[
            pl.BlockSpec((BM, HALF), lambda i: (i, 0)),
            pl.BlockSpec((BM, HALF), lambda i: (i, 0)),
        ],
        out_shape=[jax.ShapeDtypeStruct((NPAD, HALF), jnp.float32)] * 2,
    )(xp, W1, deg)


def _tc_layer(aggA, aggB, yA, yB, W, b2d, deg):
    def body(aA_ref, aB_ref, yA_ref, yB_ref, w_ref, b_ref, d_ref,
             oA_ref, oB_ref):
        dinv = _dinv_of(d_ref)
        u = jnp.concatenate(
            [aA_ref[...] + yA_ref[...], aB_ref[...] + yB_ref[...]], axis=1)
        xn = jnp.maximum(u * dinv + b_ref[...], 0.0)
        y = _mm_t(xn, w_ref[...]) * dinv
        oA_ref[...] = y[:, :HALF]
        oB_ref[...] = y[:, HALF:]

    half_spec = pl.BlockSpec((BM, HALF), lambda i: (i, 0))
    return pl.pallas_call(
        body,
        grid=(NPAD // BM,),
        in_specs=[
            half_spec, half_spec, half_spec, half_spec,
            pl.BlockSpec((H, H), lambda i: (0, 0)),
            pl.BlockSpec((1, H), lambda i: (0, 0)),
            pl.BlockSpec((BM, 16), lambda i: (i, 0)),
        ],
        out_specs=[half_spec, half_spec],
        out_shape=[jax.ShapeDtypeStruct((NPAD, HALF), jnp.float32)] * 2,
    )(aggA, aggB, yA, yB, W, b2d, deg)


def _tc_final(aggA, aggB, yA, yB, deg, b3_2d, batch3, wfc_p, bfc_p):
    nblk = NPAD // BM

    def body(aA_ref, aB_ref, yA_ref, yB_ref, d_ref, b_ref, bt_ref,
             wfc_ref, bfc_ref, out_ref, acc, cnt):
        i = pl.program_id(0)

        @pl.when(i == 0)
        def _():
            acc[...] = jnp.zeros_like(acc)
            cnt[...] = jnp.zeros_like(cnt)

        dinv = _dinv_of(d_ref)
        u = jnp.concatenate(
            [aA_ref[...] + yA_ref[...], aB_ref[...] + yB_ref[...]], axis=1)
        h = jnp.maximum(u * dinv + b_ref[...], 0.0)
        bt = jnp.reshape(bt_ref[...], (1, BM))
        ohT = (lax.broadcasted_iota(jnp.int32, (G, 1), 0) == bt
               ).astype(jnp.float32)
        acc[...] += lax.dot_general(ohT, h, (((1,), (0,)), ((), ())),
                                    preferred_element_type=jnp.float32,
                                    precision=lax.Precision.HIGHEST)
        cnt[...] += lax.dot_general(ohT, jnp.ones((BM, 128), jnp.float32),
                                    (((1,), (0,)), ((), ())),
                                    preferred_element_type=jnp.float32,
                                    precision=lax.Precision.HIGHEST)

        @pl.when(i == nblk - 1)
        def _():
            pooled = acc[...] / jnp.maximum(cnt[:, 0:1], 1.0)
            out_ref[...] = _mm_t(pooled, wfc_ref[...]) + bfc_ref[...]

    half_spec = pl.BlockSpec((BM, HALF), lambda i: (i, 0))
    return pl.pallas_call(
        body,
        grid=(nblk,),
        in_specs=[
            half_spec, half_spec, half_spec, half_spec,
            pl.BlockSpec((BM, 16), lambda i: (i, 0)),
            pl.BlockSpec((1, H), lambda i: (0, 0)),
            pl.BlockSpec((1, 1, BM), lambda i: (i, 0, 0)),
            pl.BlockSpec((128, H), lambda i: (0, 0)),
            pl.BlockSpec((1, 128), lambda i: (0, 0)),
        ],
        out_specs=pl.BlockSpec((G, 128), lambda i: (0, 0)),
        out_shape=jax.ShapeDtypeStruct((G, 128), jnp.float32),
        scratch_shapes=[
            pltpu.VMEM((G, H), jnp.float32),
            pltpu.VMEM((G, 128), jnp.float32),
        ],
    )(aggA, aggB, yA, yB, deg, b3_2d, batch3, wfc_p, bfc_p)


def kernel(x, edge_index, batch, W1, b1, W2, b2, W3, b3, Wfc, bfc):
    src = edge_index[0]
    dst = edge_index[1]
    pad_e = EP - E
    srcp = jnp.concatenate(
        [src, jnp.zeros((pad_e,), jnp.int32)]).reshape(NTILE, CH, CK)
    dstp = jnp.concatenate(
        [dst, jnp.full((pad_e,), DUMP, jnp.int32)]).reshape(NTILE, CH, CK)
    xp = jnp.concatenate([x, jnp.zeros((NPAD - N, D), jnp.float32)], axis=0)
    batchp = jnp.concatenate(
        [batch, jnp.full((NPAD - N,), G, jnp.int32)]).reshape(NPAD // BM, 1, BM)
    b1r = b1.reshape(1, H)
    b2r = b2.reshape(1, H)
    b3r = b3.reshape(1, H)
    wfc_p = jnp.zeros((128, H), jnp.float32).at[:C].set(Wfc)
    bfc_p = jnp.zeros((1, 128), jnp.float32).at[0, :C].set(bfc)

    deg0, deg1 = _sc_degree(dstp)
    deg = deg0 + deg1
    yA, yB = _tc_layer1(xp, W1, deg)
    aggA, aggB = _sc_aggregate(yA, yB, srcp, dstp)
    yA, yB = _tc_layer(aggA, aggB, yA, yB, W2, b1r, deg)
    aggA, aggB = _sc_aggregate(yA, yB, srcp, dstp)
    yA, yB = _tc_layer(aggA, aggB, yA, yB, W3, b2r, deg)
    aggA, aggB = _sc_aggregate(yA, yB, srcp, dstp)
    out = _tc_final(aggA, aggB, yA, yB, deg, b3r, batchp, wfc_p, bfc_p)
    return out[:, :C]
